# baseline (speedup 1.0000x reference)
import jax
import jax.numpy as jnp
from jax import lax
from jax.experimental import pallas as pl
from jax.experimental.pallas import tpu as pltpu
from jax.experimental.pallas import tpu_sc as plsc

N_PTS = 200000
NVIEW = 4
IMG_H = 256
IMG_W = 256
NC = 2
NS = 16
LANES = 16
VIEWS_PER_CORE = NVIEW // NC
PACK_H = IMG_H // 2
CHUNK = 12544
NPAD = NS * CHUNK
ITERS = CHUNK // LANES
NCOEF = 19
EPS = 1e-8


def _bf16_round(v):
    r = lax.bitcast_convert_type(v, jnp.uint32)
    lsb = lax.shift_right_logical(r, jnp.uint32(16)) & jnp.uint32(1)
    r = (r + jnp.uint32(0x7FFF) + lsb) & jnp.uint32(0xFFFF0000)
    return lax.bitcast_convert_type(r, jnp.float32)


def _inv4(m):
    a = [[m[:, i, j] for j in range(4)] for i in range(4)]

    def det2(r0, r1, c0, c1):
        return a[r0][c0] * a[r1][c1] - a[r0][c1] * a[r1][c0]

    s0 = det2(0, 1, 0, 1); s1 = det2(0, 1, 0, 2); s2 = det2(0, 1, 0, 3)
    s3 = det2(0, 1, 1, 2); s4 = det2(0, 1, 1, 3); s5 = det2(0, 1, 2, 3)
    c5 = det2(2, 3, 2, 3); c4 = det2(2, 3, 1, 3); c3 = det2(2, 3, 1, 2)
    c2 = det2(2, 3, 0, 3); c1 = det2(2, 3, 0, 2); c0 = det2(2, 3, 0, 1)
    det = s0 * c5 - s1 * c4 + s2 * c3 + s3 * c2 - s4 * c1 + s5 * c0
    r = 1.0 / det
    inv = [
        [(a[1][1] * c5 - a[1][2] * c4 + a[1][3] * c3) * r,
         (-a[0][1] * c5 + a[0][2] * c4 - a[0][3] * c3) * r,
         (a[3][1] * s5 - a[3][2] * s4 + a[3][3] * s3) * r,
         (-a[2][1] * s5 + a[2][2] * s4 - a[2][3] * s3) * r],
        [(-a[1][0] * c5 + a[1][2] * c2 - a[1][3] * c1) * r,
         (a[0][0] * c5 - a[0][2] * c2 + a[0][3] * c1) * r,
         (-a[3][0] * s5 + a[3][2] * s2 - a[3][3] * s1) * r,
         (a[2][0] * s5 - a[2][2] * s2 + a[2][3] * s1) * r],
        [(a[1][0] * c4 - a[1][1] * c2 + a[1][3] * c0) * r,
         (-a[0][0] * c4 + a[0][1] * c2 - a[0][3] * c0) * r,
         (a[3][0] * s4 - a[3][1] * s2 + a[3][3] * s0) * r,
         (-a[2][0] * s4 + a[2][1] * s2 - a[2][3] * s0) * r],
        [(-a[1][0] * c3 + a[1][1] * c1 - a[1][2] * c0) * r,
         (a[0][0] * c3 - a[0][1] * c1 + a[0][2] * c0) * r,
         (-a[3][0] * s3 + a[3][1] * s1 - a[3][2] * s0) * r,
         (a[2][0] * s3 - a[2][1] * s1 + a[2][2] * s0) * r],
    ]
    return jnp.stack([jnp.stack(row, axis=-1) for row in inv], axis=-2)


def _sc_body(coef_hbm, xs_hbm, ys_hbm, zs_hbm, maps_hbm, out_hbm,
             coef_v, x_v, y_v, z_v, map_a, map_b, acc_v, sem_a, sem_b):
    core = lax.axis_index("c")
    sub = lax.axis_index("s")
    base = sub * CHUNK
    bufs = (map_a, map_b)
    sems = (sem_a, sem_b)
    handles = [
        pltpu.async_copy(maps_hbm.at[core * VIEWS_PER_CORE], map_a, sem_a),
        None,
    ]
    pltpu.sync_copy(coef_hbm, coef_v)
    pltpu.sync_copy(xs_hbm.at[pl.ds(base, CHUNK)], x_v)
    pltpu.sync_copy(ys_hbm.at[pl.ds(base, CHUNK)], y_v)
    pltpu.sync_copy(zs_hbm.at[pl.ds(base, CHUNK)], z_v)

    for k in range(VIEWS_PER_CORE):
        view = core * VIEWS_PER_CORE + k
        handles[k % 2].wait()
        if k + 1 < VIEWS_PER_CORE:
            handles[(k + 1) % 2] = pltpu.async_copy(
                maps_hbm.at[view + 1], bufs[(k + 1) % 2], sems[(k + 1) % 2])
        map_v = bufs[k % 2]
        coff = view * (NCOEF * LANES)
        cv = [coef_v[pl.ds(coff + j * LANES, LANES)] for j in range(NCOEF)]
        (a0, a1, a2, a3, b0, b1, b2, b3, c0, c1, c2, c3,
         i00, i01, i02, i10, i11, i12, alpha) = cv

        def body(i, carry, k=k, map_v=map_v):
            s = i * LANES
            x = x_v[pl.ds(s, LANES)]
            y = y_v[pl.ds(s, LANES)]
            z = z_v[pl.ds(s, LANES)]
            camx = x * a0 + y * a1 + z * a2 + a3
            camy = x * b0 + y * b1 + z * b2 + b3
            camz = x * c0 + y * c1 + z * c2 + c3
            zd = camz + EPS
            xn = _bf16_round(camx / zd)
            yn = _bf16_round(camy / zd)
            u = xn * i00 + yn * i01 + i02
            v_ = xn * i10 + yn * i11 + i12
            zero = jnp.float32(0.0)
            one = jnp.float32(1.0)
            m = ((u >= zero) & (u < one) & (v_ >= zero) & (v_ < one)
                 & (camz > jnp.float32(EPS)))
            px = jnp.clip((u * jnp.float32(IMG_W)).astype(jnp.int32),
                          0, IMG_W - 1)
            py = jnp.clip((v_ * jnp.float32(IMG_H)).astype(jnp.int32),
                          0, IMG_H - 1)
            word = plsc.load_gather(map_v, [py & (PACK_H - 1), px])
            wu = lax.bitcast_convert_type(word, jnp.uint32)
            hi = py >= PACK_H
            fbits = jnp.where(hi, wu & jnp.uint32(0xFFFF0000),
                              lax.shift_left(wu, jnp.uint32(16)))
            val = lax.bitcast_convert_type(fbits, jnp.float32)
            res = jnp.where(m, val, zero) * alpha
            if k == 0:
                acc_v[pl.ds(s, LANES)] = res
            else:
                acc_v[pl.ds(s, LANES)] = acc_v[pl.ds(s, LANES)] + res
            return carry

        lax.fori_loop(0, ITERS, body, 0, unroll=2)

    pltpu.sync_copy(acc_v, out_hbm.at[pl.ds(core * NPAD + base, CHUNK)])


def kernel(gaussian_centers, score_maps, extrinsics, intrinsics, alphas):
    def bf(a):
        r = lax.bitcast_convert_type(a.astype(jnp.float32), jnp.uint32)
        lsb = lax.shift_right_logical(r, jnp.uint32(16)) & jnp.uint32(1)
        r = (r + jnp.uint32(0x7FFF) + lsb) & jnp.uint32(0xFFFF0000)
        return lax.bitcast_convert_type(r, jnp.float32)

    w2c = _inv4(extrinsics.astype(jnp.float32))
    intr = intrinsics.astype(jnp.float32)
    coef = jnp.concatenate(
        [bf(w2c[:, 0, :]), bf(w2c[:, 1, :]), bf(w2c[:, 2, :]),
         bf(intr[:, 0, :2]), intr[:, 0, 2:3],
         bf(intr[:, 1, :2]), intr[:, 1, 2:3],
         bf(alphas[:, None])], axis=1).astype(jnp.float32)
    coef_b = jnp.broadcast_to(coef[:, :, None],
                              (NVIEW, NCOEF, LANES)).reshape(-1)

    pts = bf(jnp.concatenate(
        [gaussian_centers.astype(jnp.float32),
         jnp.zeros((NPAD - N_PTS, 3), jnp.float32)], axis=0))
    xs = pts[:, 0]
    ys = pts[:, 1]
    zs = pts[:, 2]
    r = lax.bitcast_convert_type(score_maps.astype(jnp.float32), jnp.uint32)
    lsb = lax.shift_right_logical(r, jnp.uint32(16)) & jnp.uint32(1)
    rb = r + jnp.uint32(0x7FFF) + lsb
    lo = lax.shift_right_logical(rb[:, :PACK_H, :], jnp.uint32(16))
    hi = rb[:, PACK_H:, :] & jnp.uint32(0xFFFF0000)
    mwords = lax.bitcast_convert_type(lo | hi, jnp.int32)

    mesh = plsc.VectorSubcoreMesh(core_axis_name="c", subcore_axis_name="s",
                                  num_cores=NC, num_subcores=NS)
    call = pl.kernel(
        _sc_body,
        out_type=jax.ShapeDtypeStruct((NC * NPAD,), jnp.float32),
        mesh=mesh,
        compiler_params=pltpu.CompilerParams(use_tc_tiling_on_sc=False,
                                             needs_layout_passes=False),
        scratch_types=[
            pltpu.VMEM((NVIEW * NCOEF * LANES,), jnp.float32),
            pltpu.VMEM((CHUNK,), jnp.float32),
            pltpu.VMEM((CHUNK,), jnp.float32),
            pltpu.VMEM((CHUNK,), jnp.float32),
            pltpu.VMEM((PACK_H, IMG_W), jnp.int32),
            pltpu.VMEM((PACK_H, IMG_W), jnp.int32),
            pltpu.VMEM((CHUNK,), jnp.float32),
            pltpu.SemaphoreType.DMA,
            pltpu.SemaphoreType.DMA,
        ],
    )
    part = call(coef_b, xs, ys, zs, mwords)
    return part[:N_PTS] + part[NPAD:NPAD + N_PTS]


# speedup vs baseline: 1.5883x; 1.5883x over previous
import jax
import jax.numpy as jnp
from jax import lax
from jax.experimental import pallas as pl
from jax.experimental.pallas import tpu as pltpu
from jax.experimental.pallas import tpu_sc as plsc

N_PTS = 200000
NVIEW = 4
IMG_H = 256
IMG_W = 256
NC = 2
NS = 16
LANES = 16
VIEWS_PER_CORE = NVIEW // NC
PACK_H = IMG_H // 2
CHUNK = 12544
NPAD = NS * CHUNK
ITERS = CHUNK // LANES
NCOEF = 19
EPS = 1e-8


def _bf16_round(v):
    r = lax.bitcast_convert_type(v, jnp.uint32)
    lsb = lax.shift_right_logical(r, jnp.uint32(16)) & jnp.uint32(1)
    r = (r + jnp.uint32(0x7FFF) + lsb) & jnp.uint32(0xFFFF0000)
    return lax.bitcast_convert_type(r, jnp.float32)


def _inv4(m):
    a = [[m[:, i, j] for j in range(4)] for i in range(4)]

    def det2(r0, r1, c0, c1):
        return a[r0][c0] * a[r1][c1] - a[r0][c1] * a[r1][c0]

    s0 = det2(0, 1, 0, 1); s1 = det2(0, 1, 0, 2); s2 = det2(0, 1, 0, 3)
    s3 = det2(0, 1, 1, 2); s4 = det2(0, 1, 1, 3); s5 = det2(0, 1, 2, 3)
    c5 = det2(2, 3, 2, 3); c4 = det2(2, 3, 1, 3); c3 = det2(2, 3, 1, 2)
    c2 = det2(2, 3, 0, 3); c1 = det2(2, 3, 0, 2); c0 = det2(2, 3, 0, 1)
    det = s0 * c5 - s1 * c4 + s2 * c3 + s3 * c2 - s4 * c1 + s5 * c0
    r = 1.0 / det
    inv = [
        [(a[1][1] * c5 - a[1][2] * c4 + a[1][3] * c3) * r,
         (-a[0][1] * c5 + a[0][2] * c4 - a[0][3] * c3) * r,
         (a[3][1] * s5 - a[3][2] * s4 + a[3][3] * s3) * r,
         (-a[2][1] * s5 + a[2][2] * s4 - a[2][3] * s3) * r],
        [(-a[1][0] * c5 + a[1][2] * c2 - a[1][3] * c1) * r,
         (a[0][0] * c5 - a[0][2] * c2 + a[0][3] * c1) * r,
         (-a[3][0] * s5 + a[3][2] * s2 - a[3][3] * s1) * r,
         (a[2][0] * s5 - a[2][2] * s2 + a[2][3] * s1) * r],
        [(a[1][0] * c4 - a[1][1] * c2 + a[1][3] * c0) * r,
         (-a[0][0] * c4 + a[0][1] * c2 - a[0][3] * c0) * r,
         (a[3][0] * s4 - a[3][1] * s2 + a[3][3] * s0) * r,
         (-a[2][0] * s4 + a[2][1] * s2 - a[2][3] * s0) * r],
        [(-a[1][0] * c3 + a[1][1] * c1 - a[1][2] * c0) * r,
         (a[0][0] * c3 - a[0][1] * c1 + a[0][2] * c0) * r,
         (-a[3][0] * s3 + a[3][1] * s1 - a[3][2] * s0) * r,
         (a[2][0] * s3 - a[2][1] * s1 + a[2][2] * s0) * r],
    ]
    return jnp.stack([jnp.stack(row, axis=-1) for row in inv], axis=-2)


def _sc_body(coef_hbm, xs_hbm, ys_hbm, zs_hbm, maps_hbm, out_hbm,
             coef_v, x_v, y_v, z_v, map_a, map_b, acc_v, sem_a, sem_b):
    core = lax.axis_index("c")
    sub = lax.axis_index("s")
    base = sub * CHUNK
    bufs = (map_a, map_b)
    sems = (sem_a, sem_b)
    handles = [
        pltpu.async_copy(maps_hbm.at[core * VIEWS_PER_CORE], map_a, sem_a),
        None,
    ]
    pltpu.sync_copy(coef_hbm, coef_v)
    pltpu.sync_copy(xs_hbm.at[pl.ds(base, CHUNK)], x_v)
    pltpu.sync_copy(ys_hbm.at[pl.ds(base, CHUNK)], y_v)
    pltpu.sync_copy(zs_hbm.at[pl.ds(base, CHUNK)], z_v)

    for k in range(VIEWS_PER_CORE):
        view = core * VIEWS_PER_CORE + k
        handles[k % 2].wait()
        if k + 1 < VIEWS_PER_CORE:
            handles[(k + 1) % 2] = pltpu.async_copy(
                maps_hbm.at[view + 1], bufs[(k + 1) % 2], sems[(k + 1) % 2])
        map_v = bufs[k % 2]
        coff = view * (NCOEF * LANES)
        cv = [coef_v[pl.ds(coff + j * LANES, LANES)] for j in range(NCOEF)]
        (a0, a1, a2, a3, b0, b1, b2, b3, c0, c1, c2, c3,
         i00, i01, i02, i10, i11, i12, alpha) = cv

        def body(i, carry, k=k, map_v=map_v):
            s = i * LANES
            x = x_v[pl.ds(s, LANES)]
            y = y_v[pl.ds(s, LANES)]
            z = z_v[pl.ds(s, LANES)]
            camx = x * a0 + y * a1 + z * a2 + a3
            camy = x * b0 + y * b1 + z * b2 + b3
            camz = x * c0 + y * c1 + z * c2 + c3
            zd = camz + EPS
            xn = _bf16_round(camx / zd)
            yn = _bf16_round(camy / zd)
            u = xn * i00 + yn * i01 + i02
            v_ = xn * i10 + yn * i11 + i12
            zero = jnp.float32(0.0)
            one = jnp.float32(1.0)
            m = ((u >= zero) & (u < one) & (v_ >= zero) & (v_ < one)
                 & (camz > jnp.float32(EPS)))
            px = jnp.clip((u * jnp.float32(IMG_W)).astype(jnp.int32),
                          0, IMG_W - 1)
            py = jnp.clip((v_ * jnp.float32(IMG_H)).astype(jnp.int32),
                          0, IMG_H - 1)
            word = plsc.load_gather(map_v, [py & (PACK_H - 1), px])
            wu = lax.bitcast_convert_type(word, jnp.uint32)
            hi = py >= PACK_H
            fbits = jnp.where(hi, wu & jnp.uint32(0xFFFF0000),
                              lax.shift_left(wu, jnp.uint32(16)))
            val = lax.bitcast_convert_type(fbits, jnp.float32)
            res = jnp.where(m, val, zero) * alpha
            if k == 0:
                acc_v[pl.ds(s, LANES)] = res
            else:
                acc_v[pl.ds(s, LANES)] = acc_v[pl.ds(s, LANES)] + res
            return carry

        lax.fori_loop(0, ITERS, body, 0)

    pltpu.sync_copy(acc_v, out_hbm.at[pl.ds(core * NPAD + base, CHUNK)])


def kernel(gaussian_centers, score_maps, extrinsics, intrinsics, alphas):
    def bf(a):
        r = lax.bitcast_convert_type(a.astype(jnp.float32), jnp.uint32)
        lsb = lax.shift_right_logical(r, jnp.uint32(16)) & jnp.uint32(1)
        r = (r + jnp.uint32(0x7FFF) + lsb) & jnp.uint32(0xFFFF0000)
        return lax.bitcast_convert_type(r, jnp.float32)

    w2c = _inv4(extrinsics.astype(jnp.float32))
    intr = intrinsics.astype(jnp.float32)
    coef = jnp.concatenate(
        [bf(w2c[:, 0, :]), bf(w2c[:, 1, :]), bf(w2c[:, 2, :]),
         bf(intr[:, 0, :2]), intr[:, 0, 2:3],
         bf(intr[:, 1, :2]), intr[:, 1, 2:3],
         bf(alphas[:, None])], axis=1).astype(jnp.float32)
    coef_b = jnp.broadcast_to(coef[:, :, None],
                              (NVIEW, NCOEF, LANES)).reshape(-1)

    pts = bf(jnp.concatenate(
        [gaussian_centers.astype(jnp.float32),
         jnp.zeros((NPAD - N_PTS, 3), jnp.float32)], axis=0))
    xs = pts[:, 0]
    ys = pts[:, 1]
    zs = pts[:, 2]
    r = lax.bitcast_convert_type(score_maps.astype(jnp.float32), jnp.uint32)
    lsb = lax.shift_right_logical(r, jnp.uint32(16)) & jnp.uint32(1)
    rb = r + jnp.uint32(0x7FFF) + lsb
    lo = lax.shift_right_logical(rb[:, :PACK_H, :], jnp.uint32(16))
    hi = rb[:, PACK_H:, :] & jnp.uint32(0xFFFF0000)
    mwords = lax.bitcast_convert_type(lo | hi, jnp.int32)

    mesh = plsc.VectorSubcoreMesh(core_axis_name="c", subcore_axis_name="s",
                                  num_cores=NC, num_subcores=NS)
    call = pl.kernel(
        _sc_body,
        out_type=jax.ShapeDtypeStruct((NC * NPAD,), jnp.float32),
        mesh=mesh,
        compiler_params=pltpu.CompilerParams(use_tc_tiling_on_sc=False,
                                             needs_layout_passes=False),
        scratch_types=[
            pltpu.VMEM((NVIEW * NCOEF * LANES,), jnp.float32),
            pltpu.VMEM((CHUNK,), jnp.float32),
            pltpu.VMEM((CHUNK,), jnp.float32),
            pltpu.VMEM((CHUNK,), jnp.float32),
            pltpu.VMEM((PACK_H, IMG_W), jnp.int32),
            pltpu.VMEM((PACK_H, IMG_W), jnp.int32),
            pltpu.VMEM((CHUNK,), jnp.float32),
            pltpu.SemaphoreType.DMA,
            pltpu.SemaphoreType.DMA,
        ],
    )
    part = call(coef_b, xs, ys, zs, mwords)
    return part[:N_PTS] + part[NPAD:NPAD + N_PTS]


# speedup vs baseline: 1.6951x; 1.0672x over previous
import jax
import jax.numpy as jnp
from jax import lax
from jax.experimental import pallas as pl
from jax.experimental.pallas import tpu as pltpu
from jax.experimental.pallas import tpu_sc as plsc

N_PTS = 200000
NVIEW = 4
IMG_H = 256
IMG_W = 256
NC = 2
NS = 16
LANES = 16
VIEWS_PER_CORE = NVIEW // NC
PACK_H = IMG_H // 2
CHUNK = 12544
NPAD = NS * CHUNK
ITERS = CHUNK // LANES
NCOEF = 19
EPS = 1e-8


def _bf16_round(v):
    r = lax.bitcast_convert_type(v, jnp.uint32)
    lsb = lax.shift_right_logical(r, jnp.uint32(16)) & jnp.uint32(1)
    r = (r + jnp.uint32(0x7FFF) + lsb) & jnp.uint32(0xFFFF0000)
    return lax.bitcast_convert_type(r, jnp.float32)


def _inv4(m):
    a = [[m[:, i, j] for j in range(4)] for i in range(4)]

    def det2(r0, r1, c0, c1):
        return a[r0][c0] * a[r1][c1] - a[r0][c1] * a[r1][c0]

    s0 = det2(0, 1, 0, 1); s1 = det2(0, 1, 0, 2); s2 = det2(0, 1, 0, 3)
    s3 = det2(0, 1, 1, 2); s4 = det2(0, 1, 1, 3); s5 = det2(0, 1, 2, 3)
    c5 = det2(2, 3, 2, 3); c4 = det2(2, 3, 1, 3); c3 = det2(2, 3, 1, 2)
    c2 = det2(2, 3, 0, 3); c1 = det2(2, 3, 0, 2); c0 = det2(2, 3, 0, 1)
    det = s0 * c5 - s1 * c4 + s2 * c3 + s3 * c2 - s4 * c1 + s5 * c0
    r = 1.0 / det
    inv = [
        [(a[1][1] * c5 - a[1][2] * c4 + a[1][3] * c3) * r,
         (-a[0][1] * c5 + a[0][2] * c4 - a[0][3] * c3) * r,
         (a[3][1] * s5 - a[3][2] * s4 + a[3][3] * s3) * r,
         (-a[2][1] * s5 + a[2][2] * s4 - a[2][3] * s3) * r],
        [(-a[1][0] * c5 + a[1][2] * c2 - a[1][3] * c1) * r,
         (a[0][0] * c5 - a[0][2] * c2 + a[0][3] * c1) * r,
         (-a[3][0] * s5 + a[3][2] * s2 - a[3][3] * s1) * r,
         (a[2][0] * s5 - a[2][2] * s2 + a[2][3] * s1) * r],
        [(a[1][0] * c4 - a[1][1] * c2 + a[1][3] * c0) * r,
         (-a[0][0] * c4 + a[0][1] * c2 - a[0][3] * c0) * r,
         (a[3][0] * s4 - a[3][1] * s2 + a[3][3] * s0) * r,
         (-a[2][0] * s4 + a[2][1] * s2 - a[2][3] * s0) * r],
        [(-a[1][0] * c3 + a[1][1] * c1 - a[1][2] * c0) * r,
         (a[0][0] * c3 - a[0][1] * c1 + a[0][2] * c0) * r,
         (-a[3][0] * s3 + a[3][1] * s1 - a[3][2] * s0) * r,
         (a[2][0] * s3 - a[2][1] * s1 + a[2][2] * s0) * r],
    ]
    return jnp.stack([jnp.stack(row, axis=-1) for row in inv], axis=-2)


def _sc_body(coef_hbm, xs_hbm, ys_hbm, zs_hbm, maps_hbm, out_hbm,
             coef_v, x_v, y_v, z_v, map_a, map_b, acc_v, sem_a, sem_b, sem_p):
    core = lax.axis_index("c")
    sub = lax.axis_index("s")
    base = sub * CHUNK
    bufs = (map_a, map_b)
    sems = (sem_a, sem_b)
    handles = [
        pltpu.async_copy(maps_hbm.at[core * VIEWS_PER_CORE], map_a, sem_a),
        None,
    ]
    hx = pltpu.async_copy(xs_hbm.at[pl.ds(base, CHUNK)], x_v, sem_p)
    hy = pltpu.async_copy(ys_hbm.at[pl.ds(base, CHUNK)], y_v, sem_p)
    hz = pltpu.async_copy(zs_hbm.at[pl.ds(base, CHUNK)], z_v, sem_p)
    pltpu.sync_copy(coef_hbm, coef_v)
    hx.wait()
    hy.wait()
    hz.wait()

    for k in range(VIEWS_PER_CORE):
        view = core * VIEWS_PER_CORE + k
        handles[k % 2].wait()
        if k + 1 < VIEWS_PER_CORE:
            handles[(k + 1) % 2] = pltpu.async_copy(
                maps_hbm.at[view + 1], bufs[(k + 1) % 2], sems[(k + 1) % 2])
        map_v = bufs[k % 2]
        coff = view * (NCOEF * LANES)
        cv = [coef_v[pl.ds(coff + j * LANES, LANES)] for j in range(NCOEF)]
        (a0, a1, a2, a3, b0, b1, b2, b3, c0, c1, c2, c3,
         i00, i01, i02, i10, i11, i12, alpha) = cv

        def body(i, carry, k=k, map_v=map_v):
            s = i * LANES
            x = x_v[pl.ds(s, LANES)]
            y = y_v[pl.ds(s, LANES)]
            z = z_v[pl.ds(s, LANES)]
            camx = x * a0 + y * a1 + z * a2 + a3
            camy = x * b0 + y * b1 + z * b2 + b3
            camz = x * c0 + y * c1 + z * c2 + c3
            zd = camz + EPS
            xn = _bf16_round(camx / zd)
            yn = _bf16_round(camy / zd)
            u = xn * i00 + yn * i01 + i02
            v_ = xn * i10 + yn * i11 + i12
            zero = jnp.float32(0.0)
            one = jnp.float32(1.0)
            m = ((u >= zero) & (u < one) & (v_ >= zero) & (v_ < one)
                 & (camz > jnp.float32(EPS)))
            px = (u * jnp.float32(IMG_W)).astype(jnp.int32) & (IMG_W - 1)
            py = (v_ * jnp.float32(IMG_H)).astype(jnp.int32) & (IMG_H - 1)
            word = plsc.load_gather(map_v, [py & (PACK_H - 1), px])
            wu = lax.bitcast_convert_type(word, jnp.uint32)
            hi = py >= PACK_H
            fbits = jnp.where(hi, wu & jnp.uint32(0xFFFF0000),
                              lax.shift_left(wu, jnp.uint32(16)))
            val = lax.bitcast_convert_type(fbits, jnp.float32)
            res = jnp.where(m, val, zero) * alpha
            if k == 0:
                acc_v[pl.ds(s, LANES)] = res
            else:
                acc_v[pl.ds(s, LANES)] = acc_v[pl.ds(s, LANES)] + res
            return carry

        lax.fori_loop(0, ITERS, body, 0)

    pltpu.sync_copy(acc_v, out_hbm.at[pl.ds(core * NPAD + base, CHUNK)])


def kernel(gaussian_centers, score_maps, extrinsics, intrinsics, alphas):
    def bf(a):
        r = lax.bitcast_convert_type(a.astype(jnp.float32), jnp.uint32)
        lsb = lax.shift_right_logical(r, jnp.uint32(16)) & jnp.uint32(1)
        r = (r + jnp.uint32(0x7FFF) + lsb) & jnp.uint32(0xFFFF0000)
        return lax.bitcast_convert_type(r, jnp.float32)

    w2c = _inv4(extrinsics.astype(jnp.float32))
    intr = intrinsics.astype(jnp.float32)
    coef = jnp.concatenate(
        [bf(w2c[:, 0, :]), bf(w2c[:, 1, :]), bf(w2c[:, 2, :]),
         bf(intr[:, 0, :2]), intr[:, 0, 2:3],
         bf(intr[:, 1, :2]), intr[:, 1, 2:3],
         bf(alphas[:, None])], axis=1).astype(jnp.float32)
    coef_b = jnp.broadcast_to(coef[:, :, None],
                              (NVIEW, NCOEF, LANES)).reshape(-1)

    pts = bf(jnp.concatenate(
        [gaussian_centers.astype(jnp.float32),
         jnp.zeros((NPAD - N_PTS, 3), jnp.float32)], axis=0))
    xs = pts[:, 0]
    ys = pts[:, 1]
    zs = pts[:, 2]
    r = lax.bitcast_convert_type(score_maps.astype(jnp.float32), jnp.uint32)
    lsb = lax.shift_right_logical(r, jnp.uint32(16)) & jnp.uint32(1)
    rb = r + jnp.uint32(0x7FFF) + lsb
    lo = lax.shift_right_logical(rb[:, :PACK_H, :], jnp.uint32(16))
    hi = rb[:, PACK_H:, :] & jnp.uint32(0xFFFF0000)
    mwords = lax.bitcast_convert_type(lo | hi, jnp.int32)

    mesh = plsc.VectorSubcoreMesh(core_axis_name="c", subcore_axis_name="s",
                                  num_cores=NC, num_subcores=NS)
    call = pl.kernel(
        _sc_body,
        out_type=jax.ShapeDtypeStruct((NC * NPAD,), jnp.float32),
        mesh=mesh,
        compiler_params=pltpu.CompilerParams(use_tc_tiling_on_sc=False,
                                             needs_layout_passes=False),
        scratch_types=[
            pltpu.VMEM((NVIEW * NCOEF * LANES,), jnp.float32),
            pltpu.VMEM((CHUNK,), jnp.float32),
            pltpu.VMEM((CHUNK,), jnp.float32),
            pltpu.VMEM((CHUNK,), jnp.float32),
            pltpu.VMEM((PACK_H, IMG_W), jnp.int32),
            pltpu.VMEM((PACK_H, IMG_W), jnp.int32),
            pltpu.VMEM((CHUNK,), jnp.float32),
            pltpu.SemaphoreType.DMA,
            pltpu.SemaphoreType.DMA,
            pltpu.SemaphoreType.DMA,
        ],
    )
    part = call(coef_b, xs, ys, zs, mwords)
    return part[:N_PTS] + part[NPAD:NPAD + N_PTS]
